# Initial kernel scaffold; baseline (speedup 1.0000x reference)
import jax
import jax.numpy as jnp
from jax.experimental import pallas as pl


def kernel(node_features, edge_features, neighbor_indices, neighbor_masks, h, c, edge_W, edge_b, att_W1, att_b1, att_W2, att_b2, val_W, val_b):
    raise NotImplementedError("write your pallas kernel here")


# speedup vs baseline: 2.1865x; 2.1865x over previous
import functools

import jax
import jax.numpy as jnp
from jax import lax
from jax.experimental import pallas as pl
from jax.experimental.pallas import tpu as pltpu
from jax.experimental.pallas import tpu_sc as plsc

N, L, D, DE, H, A = 10000, 32, 128, 16, 8, 16
NL = N * L
BN_ = 16
NBLK = N // BN_
R = BN_ * L
NW = 32
PER_W = NL // NW
GC = 80
NCH = PER_W // GC
EPS = 1e-5


def _softplus(x):
    return jnp.maximum(x, 0.0) + jnp.log(1.0 + jnp.exp(-jnp.abs(x)))


def _gather_kernel(table_hbm, idx_hbm, out_hbm, idx_v, rows_v, sem):
    wid = lax.axis_index("s") * 2 + lax.axis_index("c")
    base = wid * PER_W
    pltpu.sync_copy(idx_hbm.at[wid], idx_v)

    def body(j, carry):
        pltpu.async_copy(table_hbm.at[idx_v.at[j]], rows_v, sem).wait()
        pltpu.sync_copy(rows_v, out_hbm.at[pl.ds(base + j * GC, GC)])
        return carry

    lax.fori_loop(0, NCH, body, 0)


def _sc_gather(table, idx2d):
    mesh = plsc.VectorSubcoreMesh(core_axis_name="c", subcore_axis_name="s")
    return pl.kernel(
        _gather_kernel,
        out_type=jax.ShapeDtypeStruct((NL, D), jnp.float32),
        mesh=mesh,
        scratch_types=[
            pltpu.VMEM((NCH, GC), jnp.int32),
            pltpu.VMEM((GC, D), jnp.float32),
            pltpu.SemaphoreType.DMA,
        ],
    )(table, idx2d)


def _t1_body(n_ref, g_ref, e_ref, wes, wen, wee, b_ref, epre_ref, acc_ref):
    i = pl.program_id(0)
    g2 = g_ref[...].reshape(R, D)
    e2 = e_ref[...].reshape(R, DE)
    zs = jnp.dot(n_ref[...], wes[...], preferred_element_type=jnp.float32)
    zn = jnp.dot(g2, wen[...], preferred_element_type=jnp.float32)
    ze = jnp.dot(e2, wee[...], preferred_element_type=jnp.float32)
    e3 = zs[:, None, :] + (zn + ze).reshape(BN_, L, DE) + b_ref[0, :]
    epre_ref[...] = e3
    s = jnp.sum(e3, axis=(0, 1))
    ss = jnp.sum(e3 * e3, axis=(0, 1))
    st = jnp.concatenate([s[None, :], ss[None, :]], axis=0)

    @pl.when(i == 0)
    def _():
        acc_ref[...] = jnp.zeros_like(acc_ref)

    acc_ref[...] += st


def _t3_body(n_ref, g_ref, ep_ref, e_ref, acc1, w1s, w1n, w1e, b1, wvs, wvn,
             wve, bv, w2b, b2, exp_ref, eo_ref, w_ref, acc_ref):
    i = pl.program_id(0)
    m1 = acc1[0, :] / NL
    rs1 = lax.rsqrt(acc1[1, :] / NL - m1 * m1 + EPS)
    eo3 = _softplus(e_ref[...] + (ep_ref[...] - m1) * rs1)
    eo_ref[...] = eo3
    eo2 = eo3.reshape(R, DE)
    g2 = g_ref[...].reshape(R, D)
    nblk = n_ref[...]
    ps = jnp.dot(nblk, w1s[...], preferred_element_type=jnp.float32)
    vs = jnp.dot(nblk, wvs[...], preferred_element_type=jnp.float32)
    hid3 = _softplus(
        (jnp.dot(g2, w1n[...], preferred_element_type=jnp.float32)
         + jnp.dot(eo2, w1e[...], preferred_element_type=jnp.float32)
         ).reshape(BN_, L, H * A) + ps[:, None, :] + b1[0, :])
    lg = (jnp.dot(hid3.reshape(R, H * A), w2b[...],
                  preferred_element_type=jnp.float32) + b2[0, :]
          ).reshape(BN_, L, H)
    mx = jnp.max(lg, axis=1, keepdims=True)
    ex = jnp.exp(lg - mx)
    a3 = ex / jnp.sum(ex, axis=1, keepdims=True)
    aexp = jnp.dot(a3.reshape(R, H), exp_ref[...],
                   preferred_element_type=jnp.float32).reshape(BN_, L, H * A)
    v3 = (jnp.dot(g2, wvn[...], preferred_element_type=jnp.float32)
          + jnp.dot(eo2, wve[...], preferred_element_type=jnp.float32)
          ).reshape(BN_, L, H * A) + vs[:, None, :] + bv[0, :]
    w3 = aexp * v3
    w_ref[...] = w3
    s = jnp.sum(w3, axis=(0, 1))
    ss = jnp.sum(w3 * w3, axis=(0, 1))
    st = jnp.concatenate([s[None, :], ss[None, :]], axis=0)

    @pl.when(i == 0)
    def _():
        acc_ref[...] = jnp.zeros_like(acc_ref)

    acc_ref[...] += st


def _t4_body(w_ref, acc3, cat_ref, acc_ref):
    i = pl.program_id(0)
    m = acc3[0, :] / NL
    rs = lax.rsqrt(acc3[1, :] / NL - m * m + EPS)
    hv = _softplus((w_ref[...] - m) * rs)
    cat = jnp.sum(hv, axis=1)
    cat_ref[...] = cat
    s = jnp.sum(cat, axis=0)
    ss = jnp.sum(cat * cat, axis=0)
    st = jnp.concatenate([s[None, :], ss[None, :]], axis=0)

    @pl.when(i == 0)
    def _():
        acc_ref[...] = jnp.zeros_like(acc_ref)

    acc_ref[...] += st


def _t5_body(n_ref, cat_ref, acc4, out_ref):
    m = acc4[0, :] / N
    rs = lax.rsqrt(acc4[1, :] / N - m * m + EPS)
    out_ref[...] = n_ref[...] + (cat_ref[...] - m) * rs


def _full(shape):
    nd = len(shape)
    return pl.BlockSpec(shape, lambda i: (0,) * nd)


_SEQ = pltpu.CompilerParams(dimension_semantics=("arbitrary",))


def _tc_pipeline(node_features, gathered3, edge_features, edge_W, edge_b,
                 att_W1, att_b1, att_W2, att_b2, val_W, val_b):
    f32 = jnp.float32
    wes, wen, wee = edge_W[:D], edge_W[D:2 * D], edge_W[2 * D:]
    w1cat = jnp.transpose(att_W1, (1, 0, 2)).reshape(2 * D + DE, H * A)
    w1s, w1n, w1e = w1cat[:D], w1cat[D:2 * D], w1cat[2 * D:]
    wvcat = jnp.transpose(val_W, (1, 0, 2)).reshape(2 * D + DE, H * A)
    wvs, wvn, wve = wvcat[:D], wvcat[D:2 * D], wvcat[2 * D:]
    b1 = att_b1.reshape(1, H * A)
    bv = val_b.reshape(1, H * A)
    w2b = jax.scipy.linalg.block_diag(*[att_W2[hh] for hh in range(H)])
    b2 = att_b2.reshape(1, H)
    expander = jnp.repeat(jnp.eye(H, dtype=f32), A, axis=1)
    eb = edge_b.reshape(1, DE)

    nspec = pl.BlockSpec((BN_, D), lambda i: (i, 0))
    gspec = pl.BlockSpec((BN_, L, D), lambda i: (i, 0, 0))
    espec = pl.BlockSpec((BN_, L, DE), lambda i: (i, 0, 0))

    epre, acc1 = pl.pallas_call(
        _t1_body,
        grid=(NBLK,),
        in_specs=[nspec, gspec, espec, _full((D, DE)), _full((D, DE)),
                  _full((DE, DE)), _full((1, DE))],
        out_specs=[espec, _full((2, DE))],
        out_shape=[jax.ShapeDtypeStruct((N, L, DE), f32),
                   jax.ShapeDtypeStruct((2, DE), f32)],
        compiler_params=_SEQ,
    )(node_features, gathered3, edge_features, wes, wen, wee, eb)

    edge_out, weighted, acc3 = pl.pallas_call(
        _t3_body,
        grid=(NBLK,),
        in_specs=[nspec, gspec, espec, espec, _full((2, DE)),
                  _full((D, H * A)), _full((D, H * A)), _full((DE, H * A)),
                  _full((1, H * A)),
                  _full((D, H * A)), _full((D, H * A)), _full((DE, H * A)),
                  _full((1, H * A)),
                  _full((H * A, H)), _full((1, H)), _full((H, H * A))],
        out_specs=[espec, gspec, _full((2, H * A))],
        out_shape=[jax.ShapeDtypeStruct((N, L, DE), f32),
                   jax.ShapeDtypeStruct((N, L, H * A), f32),
                   jax.ShapeDtypeStruct((2, H * A), f32)],
        compiler_params=_SEQ,
    )(node_features, gathered3, epre, edge_features, acc1, w1s, w1n, w1e, b1,
      wvs, wvn, wve, bv, w2b, b2, expander)

    cat, acc4 = pl.pallas_call(
        _t4_body,
        grid=(NBLK,),
        in_specs=[gspec, _full((2, H * A))],
        out_specs=[nspec, _full((2, H * A))],
        out_shape=[jax.ShapeDtypeStruct((N, H * A), f32),
                   jax.ShapeDtypeStruct((2, H * A), f32)],
        compiler_params=_SEQ,
    )(weighted, acc3)

    n5spec = pl.BlockSpec((1000, D), lambda i: (i, 0))
    atom_out = pl.pallas_call(
        _t5_body,
        grid=(10,),
        in_specs=[n5spec, n5spec, _full((2, H * A))],
        out_specs=n5spec,
        out_shape=jax.ShapeDtypeStruct((N, D), f32),
        compiler_params=_SEQ,
    )(node_features, cat, acc4)

    return atom_out, edge_out


def kernel(node_features, edge_features, neighbor_indices, neighbor_masks,
           h, c, edge_W, edge_b, att_W1, att_b1, att_W2, att_b2, val_W,
           val_b):
    idx2d = neighbor_indices.astype(jnp.int32).reshape(NW, NCH, GC)
    gathered = _sc_gather(node_features, idx2d)
    gathered3 = gathered.reshape(N, L, D)
    atom_out, edge_out = _tc_pipeline(
        node_features, gathered3, edge_features, edge_W, edge_b,
        att_W1, att_b1, att_W2, att_b2, val_W, val_b)
    return (atom_out, edge_out, h, c)


# speedup vs baseline: 2.2346x; 1.0220x over previous
import functools

import jax
import jax.numpy as jnp
from jax import lax
from jax.experimental import pallas as pl
from jax.experimental.pallas import tpu as pltpu
from jax.experimental.pallas import tpu_sc as plsc

N, L, D, DE, H, A = 10000, 32, 128, 16, 8, 16
NL = N * L
BN_ = 16
NBLK = N // BN_
R = BN_ * L
NW = 32
PER_W = NL // NW
GC = 80
NCH = PER_W // GC
EPS = 1e-5


def _softplus(x):
    return jnp.maximum(x, 0.0) + jnp.log(1.0 + jnp.exp(-jnp.abs(x)))


def _gather_kernel(table_hbm, idx_hbm, out_hbm, idx_v, rows_v, sem):
    wid = lax.axis_index("s") * 2 + lax.axis_index("c")
    base = wid * PER_W
    pltpu.sync_copy(idx_hbm.at[wid], idx_v)

    def body(j, carry):
        pltpu.async_copy(table_hbm.at[idx_v.at[j]], rows_v, sem).wait()
        pltpu.sync_copy(rows_v, out_hbm.at[pl.ds(base + j * GC, GC)])
        return carry

    lax.fori_loop(0, NCH, body, 0)


def _sc_gather(table, idx2d):
    mesh = plsc.VectorSubcoreMesh(core_axis_name="c", subcore_axis_name="s")
    return pl.kernel(
        _gather_kernel,
        out_type=jax.ShapeDtypeStruct((NL, D), jnp.float32),
        mesh=mesh,
        scratch_types=[
            pltpu.VMEM((NCH, GC), jnp.int32),
            pltpu.VMEM((GC, D), jnp.float32),
            pltpu.SemaphoreType.DMA,
        ],
    )(table, idx2d)


def _t1_body(n_ref, g_ref, e_ref, wes, wen, wee, b_ref, acc_ref):
    i = pl.program_id(0)
    g2 = g_ref[...].reshape(R, D)
    e2 = e_ref[...].reshape(R, DE)
    zs = jnp.dot(n_ref[...], wes[...], preferred_element_type=jnp.float32)
    zn = jnp.dot(g2, wen[...], preferred_element_type=jnp.float32)
    ze = jnp.dot(e2, wee[...], preferred_element_type=jnp.float32)
    e3 = zs[:, None, :] + (zn + ze).reshape(BN_, L, DE) + b_ref[0, :]
    s = jnp.sum(e3, axis=(0, 1))
    ss = jnp.sum(e3 * e3, axis=(0, 1))
    st = jnp.concatenate([s[None, :], ss[None, :]], axis=0)

    @pl.when(i == 0)
    def _():
        acc_ref[...] = jnp.zeros_like(acc_ref)

    acc_ref[...] += st


def _t3_body(n_ref, g_ref, e_ref, acc1, wes, wen, wee, eb, w1s, w1n, w1e, b1,
             wvs, wvn, wve, bv, w2b, b2, exp_ref, eo_ref, w_ref, acc_ref):
    i = pl.program_id(0)
    g2 = g_ref[...].reshape(R, D)
    nblk = n_ref[...]
    e2 = e_ref[...].reshape(R, DE)
    zs = jnp.dot(nblk, wes[...], preferred_element_type=jnp.float32)
    zn = jnp.dot(g2, wen[...], preferred_element_type=jnp.float32)
    ze = jnp.dot(e2, wee[...], preferred_element_type=jnp.float32)
    ep3 = zs[:, None, :] + (zn + ze).reshape(BN_, L, DE) + eb[0, :]
    m1 = acc1[0, :] / NL
    rs1 = lax.rsqrt(acc1[1, :] / NL - m1 * m1 + EPS)
    eo3 = _softplus(e_ref[...] + (ep3 - m1) * rs1)
    eo_ref[...] = eo3
    eo2 = eo3.reshape(R, DE)
    ps = jnp.dot(nblk, w1s[...], preferred_element_type=jnp.float32)
    vs = jnp.dot(nblk, wvs[...], preferred_element_type=jnp.float32)
    hid3 = _softplus(
        (jnp.dot(g2, w1n[...], preferred_element_type=jnp.float32)
         + jnp.dot(eo2, w1e[...], preferred_element_type=jnp.float32)
         ).reshape(BN_, L, H * A) + ps[:, None, :] + b1[0, :])
    lg = (jnp.dot(hid3.reshape(R, H * A), w2b[...],
                  preferred_element_type=jnp.float32) + b2[0, :]
          ).reshape(BN_, L, H)
    mx = jnp.max(lg, axis=1, keepdims=True)
    ex = jnp.exp(lg - mx)
    a3 = ex / jnp.sum(ex, axis=1, keepdims=True)
    aexp = jnp.dot(a3.reshape(R, H), exp_ref[...],
                   preferred_element_type=jnp.float32).reshape(BN_, L, H * A)
    v3 = (jnp.dot(g2, wvn[...], preferred_element_type=jnp.float32)
          + jnp.dot(eo2, wve[...], preferred_element_type=jnp.float32)
          ).reshape(BN_, L, H * A) + vs[:, None, :] + bv[0, :]
    w3 = aexp * v3
    w_ref[...] = w3.astype(jnp.bfloat16)
    s = jnp.sum(w3, axis=(0, 1))
    ss = jnp.sum(w3 * w3, axis=(0, 1))
    st = jnp.concatenate([s[None, :], ss[None, :]], axis=0)

    @pl.when(i == 0)
    def _():
        acc_ref[...] = jnp.zeros_like(acc_ref)

    acc_ref[...] += st


def _t4_body(w_ref, acc3, cat_ref, acc_ref):
    i = pl.program_id(0)
    m = acc3[0, :] / NL
    rs = lax.rsqrt(acc3[1, :] / NL - m * m + EPS)
    hv = _softplus((w_ref[...].astype(jnp.float32) - m) * rs)
    cat = jnp.sum(hv, axis=1)
    cat_ref[...] = cat
    s = jnp.sum(cat, axis=0)
    ss = jnp.sum(cat * cat, axis=0)
    st = jnp.concatenate([s[None, :], ss[None, :]], axis=0)

    @pl.when(i == 0)
    def _():
        acc_ref[...] = jnp.zeros_like(acc_ref)

    acc_ref[...] += st


def _t5_body(n_ref, cat_ref, acc4, out_ref):
    m = acc4[0, :] / N
    rs = lax.rsqrt(acc4[1, :] / N - m * m + EPS)
    out_ref[...] = n_ref[...] + (cat_ref[...] - m) * rs


def _full(shape):
    nd = len(shape)
    return pl.BlockSpec(shape, lambda i: (0,) * nd)


_SEQ = pltpu.CompilerParams(dimension_semantics=("arbitrary",))


def _tc_pipeline(node_features, gathered3, edge_features, edge_W, edge_b,
                 att_W1, att_b1, att_W2, att_b2, val_W, val_b):
    f32 = jnp.float32
    wes, wen, wee = edge_W[:D], edge_W[D:2 * D], edge_W[2 * D:]
    w1cat = jnp.transpose(att_W1, (1, 0, 2)).reshape(2 * D + DE, H * A)
    w1s, w1n, w1e = w1cat[:D], w1cat[D:2 * D], w1cat[2 * D:]
    wvcat = jnp.transpose(val_W, (1, 0, 2)).reshape(2 * D + DE, H * A)
    wvs, wvn, wve = wvcat[:D], wvcat[D:2 * D], wvcat[2 * D:]
    b1 = att_b1.reshape(1, H * A)
    bv = val_b.reshape(1, H * A)
    w2b = jax.scipy.linalg.block_diag(*[att_W2[hh] for hh in range(H)])
    b2 = att_b2.reshape(1, H)
    expander = jnp.repeat(jnp.eye(H, dtype=f32), A, axis=1)
    eb = edge_b.reshape(1, DE)

    nspec = pl.BlockSpec((BN_, D), lambda i: (i, 0))
    gspec = pl.BlockSpec((BN_, L, D), lambda i: (i, 0, 0))
    espec = pl.BlockSpec((BN_, L, DE), lambda i: (i, 0, 0))

    acc1 = pl.pallas_call(
        _t1_body,
        grid=(NBLK,),
        in_specs=[nspec, gspec, espec, _full((D, DE)), _full((D, DE)),
                  _full((DE, DE)), _full((1, DE))],
        out_specs=_full((2, DE)),
        out_shape=jax.ShapeDtypeStruct((2, DE), f32),
        compiler_params=_SEQ,
    )(node_features, gathered3, edge_features, wes, wen, wee, eb)

    edge_out, weighted, acc3 = pl.pallas_call(
        _t3_body,
        grid=(NBLK,),
        in_specs=[nspec, gspec, espec, _full((2, DE)),
                  _full((D, DE)), _full((D, DE)), _full((DE, DE)),
                  _full((1, DE)),
                  _full((D, H * A)), _full((D, H * A)), _full((DE, H * A)),
                  _full((1, H * A)),
                  _full((D, H * A)), _full((D, H * A)), _full((DE, H * A)),
                  _full((1, H * A)),
                  _full((H * A, H)), _full((1, H)), _full((H, H * A))],
        out_specs=[espec, gspec, _full((2, H * A))],
        out_shape=[jax.ShapeDtypeStruct((N, L, DE), f32),
                   jax.ShapeDtypeStruct((N, L, H * A), jnp.bfloat16),
                   jax.ShapeDtypeStruct((2, H * A), f32)],
        compiler_params=_SEQ,
    )(node_features, gathered3, edge_features, acc1, wes, wen, wee, eb,
      w1s, w1n, w1e, b1, wvs, wvn, wve, bv, w2b, b2, expander)

    cat, acc4 = pl.pallas_call(
        _t4_body,
        grid=(NBLK,),
        in_specs=[gspec, _full((2, H * A))],
        out_specs=[nspec, _full((2, H * A))],
        out_shape=[jax.ShapeDtypeStruct((N, H * A), f32),
                   jax.ShapeDtypeStruct((2, H * A), f32)],
        compiler_params=_SEQ,
    )(weighted, acc3)

    n5spec = pl.BlockSpec((1000, D), lambda i: (i, 0))
    atom_out = pl.pallas_call(
        _t5_body,
        grid=(10,),
        in_specs=[n5spec, n5spec, _full((2, H * A))],
        out_specs=n5spec,
        out_shape=jax.ShapeDtypeStruct((N, D), f32),
        compiler_params=_SEQ,
    )(node_features, cat, acc4)

    return atom_out, edge_out


def kernel(node_features, edge_features, neighbor_indices, neighbor_masks,
           h, c, edge_W, edge_b, att_W1, att_b1, att_W2, att_b2, val_W,
           val_b):
    idx2d = neighbor_indices.astype(jnp.int32).reshape(NW, NCH, GC)
    gathered = _sc_gather(node_features, idx2d)
    gathered3 = gathered.reshape(N, L, D)
    atom_out, edge_out = _tc_pipeline(
        node_features, gathered3, edge_features, edge_W, edge_b,
        att_W1, att_b1, att_W2, att_b2, val_W, val_b)
    return (atom_out, edge_out, h, c)


# speedup vs baseline: 2.2346x; 1.0000x over previous
import functools

import jax
import jax.numpy as jnp
from jax import lax
from jax.experimental import pallas as pl
from jax.experimental.pallas import tpu as pltpu
from jax.experimental.pallas import tpu_sc as plsc

N, L, D, DE, H, A = 10000, 32, 128, 16, 8, 16
NL = N * L
BN_ = 16
NBLK = N // BN_
R = BN_ * L
NW = 32
PER_W = NL // NW
GC = 80
NCH = PER_W // GC
EPS = 1e-5


def _softplus(x):
    return jnp.maximum(x, 0.0) + jnp.log(1.0 + jnp.exp(-jnp.abs(x)))


GD = D // 2


def _gather_kernel(table_hbm, idx_hbm, out_hbm, idx_v, rows_v, sem):
    wid = lax.axis_index("s") * 2 + lax.axis_index("c")
    base = wid * PER_W
    pltpu.sync_copy(idx_hbm.at[wid], idx_v)

    def body(j, carry):
        pltpu.async_copy(table_hbm.at[idx_v.at[j]], rows_v, sem).wait()
        pltpu.sync_copy(rows_v, out_hbm.at[pl.ds(base + j * GC, GC)])
        return carry

    lax.fori_loop(0, NCH, body, 0)


def _sc_gather(table, idx2d):
    mesh = plsc.VectorSubcoreMesh(core_axis_name="c", subcore_axis_name="s")
    return pl.kernel(
        _gather_kernel,
        out_type=jax.ShapeDtypeStruct((NL, D), jnp.float32),
        mesh=mesh,
        scratch_types=[
            pltpu.VMEM((NCH, GC), jnp.int32),
            pltpu.VMEM((GC, D), jnp.float32),
            pltpu.SemaphoreType.DMA,
        ],
    )(table, idx2d)


def _t1_body(n_ref, g_ref, e_ref, wes, wen, wee, b_ref, acc_ref):
    i = pl.program_id(0)
    g2 = g_ref[...].reshape(R, D).astype(jnp.bfloat16)
    e2 = e_ref[...].reshape(R, DE)
    zs = jnp.dot(n_ref[...], wes[...], preferred_element_type=jnp.float32)
    zn = jnp.dot(g2, wen[...], preferred_element_type=jnp.float32)
    ze = jnp.dot(e2, wee[...], preferred_element_type=jnp.float32)
    e3 = zs[:, None, :] + (zn + ze).reshape(BN_, L, DE) + b_ref[0, :]
    s = jnp.sum(e3, axis=(0, 1))
    ss = jnp.sum(e3 * e3, axis=(0, 1))
    st = jnp.concatenate([s[None, :], ss[None, :]], axis=0)

    @pl.when(i == 0)
    def _():
        acc_ref[...] = jnp.zeros_like(acc_ref)

    acc_ref[...] += st


def _t3_body(n_ref, g_ref, e_ref, acc1, wes, wen, wee, eb, w1s, w1n, w1e, b1,
             wvs, wvn, wve, bv, w2b, b2, exp_ref, eo_ref, w_ref, acc_ref):
    i = pl.program_id(0)
    g2 = g_ref[...].reshape(R, D).astype(jnp.bfloat16)
    nblk = n_ref[...]
    e2 = e_ref[...].reshape(R, DE)
    zs = jnp.dot(nblk, wes[...], preferred_element_type=jnp.float32)
    zn = jnp.dot(g2, wen[...], preferred_element_type=jnp.float32)
    ze = jnp.dot(e2, wee[...], preferred_element_type=jnp.float32)
    ep3 = zs[:, None, :] + (zn + ze).reshape(BN_, L, DE) + eb[0, :]
    m1 = acc1[0, :] / NL
    rs1 = lax.rsqrt(acc1[1, :] / NL - m1 * m1 + EPS)
    eo3 = _softplus(e_ref[...] + (ep3 - m1) * rs1)
    eo_ref[...] = eo3
    eo2 = eo3.reshape(R, DE)
    ps = jnp.dot(nblk, w1s[...], preferred_element_type=jnp.float32)
    vs = jnp.dot(nblk, wvs[...], preferred_element_type=jnp.float32)
    hid3 = _softplus(
        (jnp.dot(g2, w1n[...], preferred_element_type=jnp.float32)
         + jnp.dot(eo2, w1e[...], preferred_element_type=jnp.float32)
         ).reshape(BN_, L, H * A) + ps[:, None, :] + b1[0, :])
    lg = (jnp.dot(hid3.reshape(R, H * A), w2b[...],
                  preferred_element_type=jnp.float32) + b2[0, :]
          ).reshape(BN_, L, H)
    mx = jnp.max(lg, axis=1, keepdims=True)
    ex = jnp.exp(lg - mx)
    a3 = ex / jnp.sum(ex, axis=1, keepdims=True)
    aexp = jnp.dot(a3.reshape(R, H), exp_ref[...],
                   preferred_element_type=jnp.float32).reshape(BN_, L, H * A)
    v3 = (jnp.dot(g2, wvn[...], preferred_element_type=jnp.float32)
          + jnp.dot(eo2, wve[...], preferred_element_type=jnp.float32)
          ).reshape(BN_, L, H * A) + vs[:, None, :] + bv[0, :]
    w3 = aexp * v3
    w_ref[...] = w3.astype(jnp.bfloat16)
    s = jnp.sum(w3, axis=(0, 1))
    ss = jnp.sum(w3 * w3, axis=(0, 1))
    st = jnp.concatenate([s[None, :], ss[None, :]], axis=0)

    @pl.when(i == 0)
    def _():
        acc_ref[...] = jnp.zeros_like(acc_ref)

    acc_ref[...] += st


def _t4_body(w_ref, acc3, cat_ref, acc_ref):
    i = pl.program_id(0)
    m = acc3[0, :] / NL
    rs = lax.rsqrt(acc3[1, :] / NL - m * m + EPS)
    hv = _softplus((w_ref[...].astype(jnp.float32) - m) * rs)
    cat = jnp.sum(hv, axis=1)
    cat_ref[...] = cat
    s = jnp.sum(cat, axis=0)
    ss = jnp.sum(cat * cat, axis=0)
    st = jnp.concatenate([s[None, :], ss[None, :]], axis=0)

    @pl.when(i == 0)
    def _():
        acc_ref[...] = jnp.zeros_like(acc_ref)

    acc_ref[...] += st


def _t5_body(n_ref, cat_ref, acc4, out_ref):
    m = acc4[0, :] / N
    rs = lax.rsqrt(acc4[1, :] / N - m * m + EPS)
    out_ref[...] = n_ref[...] + (cat_ref[...] - m) * rs


def _full(shape):
    nd = len(shape)
    return pl.BlockSpec(shape, lambda i: (0,) * nd)


_SEQ = pltpu.CompilerParams(dimension_semantics=("arbitrary",))


def _tc_pipeline(node_features, gathered3, edge_features, edge_W, edge_b,
                 att_W1, att_b1, att_W2, att_b2, val_W, val_b):
    f32 = jnp.float32
    bf16 = jnp.bfloat16
    wes, wee = edge_W[:D], edge_W[2 * D:]
    wen = edge_W[D:2 * D].astype(bf16)
    w1cat = jnp.transpose(att_W1, (1, 0, 2)).reshape(2 * D + DE, H * A)
    w1s, w1e = w1cat[:D], w1cat[2 * D:]
    w1n = w1cat[D:2 * D].astype(bf16)
    wvcat = jnp.transpose(val_W, (1, 0, 2)).reshape(2 * D + DE, H * A)
    wvs, wve = wvcat[:D], wvcat[2 * D:]
    wvn = wvcat[D:2 * D].astype(bf16)
    b1 = att_b1.reshape(1, H * A)
    bv = val_b.reshape(1, H * A)
    w2b = jax.scipy.linalg.block_diag(*[att_W2[hh] for hh in range(H)])
    b2 = att_b2.reshape(1, H)
    expander = jnp.repeat(jnp.eye(H, dtype=f32), A, axis=1)
    eb = edge_b.reshape(1, DE)

    nspec = pl.BlockSpec((BN_, D), lambda i: (i, 0))
    gspec = pl.BlockSpec((BN_, L, D), lambda i: (i, 0, 0))
    espec = pl.BlockSpec((BN_, L, DE), lambda i: (i, 0, 0))

    acc1 = pl.pallas_call(
        _t1_body,
        grid=(NBLK,),
        in_specs=[nspec, gspec, espec, _full((D, DE)), _full((D, DE)),
                  _full((DE, DE)), _full((1, DE))],
        compiler_params=_SEQ,
        out_specs=_full((2, DE)),
        out_shape=jax.ShapeDtypeStruct((2, DE), f32),
    )(node_features, gathered3, edge_features, wes, wen, wee, eb)

    edge_out, weighted, acc3 = pl.pallas_call(
        _t3_body,
        grid=(NBLK,),
        in_specs=[nspec, gspec, espec, _full((2, DE)),
                  _full((D, DE)), _full((D, DE)), _full((DE, DE)),
                  _full((1, DE)),
                  _full((D, H * A)), _full((D, H * A)), _full((DE, H * A)),
                  _full((1, H * A)),
                  _full((D, H * A)), _full((D, H * A)), _full((DE, H * A)),
                  _full((1, H * A)),
                  _full((H * A, H)), _full((1, H)), _full((H, H * A))],
        out_specs=[espec, gspec, _full((2, H * A))],
        out_shape=[jax.ShapeDtypeStruct((N, L, DE), f32),
                   jax.ShapeDtypeStruct((N, L, H * A), jnp.bfloat16),
                   jax.ShapeDtypeStruct((2, H * A), f32)],
        compiler_params=_SEQ,
    )(node_features, gathered3, edge_features, acc1, wes, wen, wee, eb,
      w1s, w1n, w1e, b1, wvs, wvn, wve, bv, w2b, b2, expander)

    cat, acc4 = pl.pallas_call(
        _t4_body,
        grid=(NBLK,),
        in_specs=[gspec, _full((2, H * A))],
        out_specs=[nspec, _full((2, H * A))],
        out_shape=[jax.ShapeDtypeStruct((N, H * A), f32),
                   jax.ShapeDtypeStruct((2, H * A), f32)],
        compiler_params=_SEQ,
    )(weighted, acc3)

    n5spec = pl.BlockSpec((1000, D), lambda i: (i, 0))
    atom_out = pl.pallas_call(
        _t5_body,
        grid=(10,),
        in_specs=[n5spec, n5spec, _full((2, H * A))],
        out_specs=n5spec,
        out_shape=jax.ShapeDtypeStruct((N, D), f32),
        compiler_params=_SEQ,
    )(node_features, cat, acc4)

    return atom_out, edge_out


def kernel(node_features, edge_features, neighbor_indices, neighbor_masks,
           h, c, edge_W, edge_b, att_W1, att_b1, att_W2, att_b2, val_W,
           val_b):
    idx2d = neighbor_indices.astype(jnp.int32).reshape(NW, NCH, GC)
    gathered3 = _sc_gather(node_features, idx2d).reshape(N, L, D)
    atom_out, edge_out = _tc_pipeline(
        node_features, gathered3, edge_features, edge_W, edge_b,
        att_W1, att_b1, att_W2, att_b2, val_W, val_b)
    return (atom_out, edge_out, h, c)


# speedup vs baseline: 4.0536x; 1.8140x over previous
import functools

import jax
import jax.numpy as jnp
from jax import lax
from jax.experimental import pallas as pl
from jax.experimental.pallas import tpu as pltpu
from jax.experimental.pallas import tpu_sc as plsc

N, L, D, DE, H, A = 10000, 32, 128, 16, 8, 16
NL = N * L
BN_ = 80
NBLK = N // BN_
R = BN_ * L
NW = 32
PER_W = NL // NW
GC = 80
NCH = PER_W // GC
EPS = 1e-5


def _softplus(x):
    return jnp.maximum(x, 0.0) + jnp.log(1.0 + jnp.exp(-jnp.abs(x)))


GD = D // 2


def _gather_kernel(table_hbm, idx_hbm, out_hbm, idx_v, rows_v, sem):
    wid = lax.axis_index("s") * 2 + lax.axis_index("c")
    base = wid * PER_W
    pltpu.sync_copy(idx_hbm.at[wid], idx_v)

    def body(j, carry):
        pltpu.async_copy(table_hbm.at[idx_v.at[j]], rows_v, sem).wait()
        pltpu.sync_copy(rows_v, out_hbm.at[pl.ds(base + j * GC, GC)])
        return carry

    lax.fori_loop(0, NCH, body, 0)


def _sc_gather(table, idx2d):
    mesh = plsc.VectorSubcoreMesh(core_axis_name="c", subcore_axis_name="s")
    return pl.kernel(
        _gather_kernel,
        out_type=jax.ShapeDtypeStruct((NL, D), jnp.float32),
        mesh=mesh,
        scratch_types=[
            pltpu.VMEM((NCH, GC), jnp.int32),
            pltpu.VMEM((GC, D), jnp.float32),
            pltpu.SemaphoreType.DMA,
        ],
    )(table, idx2d)


def _t1_body(n_ref, g_ref, e_ref, wes, wen, wee, b_ref, acc_ref):
    i = pl.program_id(0)
    bf16 = jnp.bfloat16
    g2 = g_ref[...].reshape(R, D).astype(bf16)
    e2 = e_ref[...].reshape(R, DE).astype(bf16)
    zs = jnp.dot(n_ref[...].astype(bf16), wes[...],
                 preferred_element_type=jnp.float32)
    zn = jnp.dot(g2, wen[...], preferred_element_type=jnp.float32)
    ze = jnp.dot(e2, wee[...], preferred_element_type=jnp.float32)
    e3 = zs[:, None, :] + (zn + ze).reshape(BN_, L, DE) + b_ref[0, :]
    s = jnp.sum(e3, axis=(0, 1))
    ss = jnp.sum(e3 * e3, axis=(0, 1))
    st = jnp.concatenate([s[None, :], ss[None, :]], axis=0)

    @pl.when(i == 0)
    def _():
        acc_ref[...] = jnp.zeros_like(acc_ref)

    acc_ref[...] += st


def _t3_body(n_ref, g_ref, e_ref, acc1, wes, wen, wee, eb, w1s, w1n, w1e, b1,
             wvs, wvn, wve, bv, w2b, b2, exp_ref, eo_ref, w_ref, acc_ref):
    i = pl.program_id(0)
    bf16 = jnp.bfloat16
    g2 = g_ref[...].reshape(R, D).astype(bf16)
    nblk = n_ref[...].astype(bf16)
    e2 = e_ref[...].reshape(R, DE).astype(bf16)
    zs = jnp.dot(nblk, wes[...], preferred_element_type=jnp.float32)
    zn = jnp.dot(g2, wen[...], preferred_element_type=jnp.float32)
    ze = jnp.dot(e2, wee[...], preferred_element_type=jnp.float32)
    ep3 = zs[:, None, :] + (zn + ze).reshape(BN_, L, DE) + eb[0, :]
    m1 = acc1[0, :] / NL
    rs1 = lax.rsqrt(acc1[1, :] / NL - m1 * m1 + EPS)
    eo3 = _softplus(e_ref[...] + (ep3 - m1) * rs1)
    eo_ref[...] = eo3
    eo2 = eo3.reshape(R, DE).astype(bf16)
    ps = jnp.dot(nblk, w1s[...], preferred_element_type=jnp.float32)
    vs = jnp.dot(nblk, wvs[...], preferred_element_type=jnp.float32)
    hid3 = _softplus(
        (jnp.dot(g2, w1n[...], preferred_element_type=jnp.float32)
         + jnp.dot(eo2, w1e[...], preferred_element_type=jnp.float32)
         ).reshape(BN_, L, H * A) + ps[:, None, :] + b1[0, :])
    lg = (jnp.dot(hid3.reshape(R, H * A).astype(bf16), w2b[...],
                  preferred_element_type=jnp.float32) + b2[0, :]
          ).reshape(BN_, L, H)
    mx = jnp.max(lg, axis=1, keepdims=True)
    ex = jnp.exp(lg - mx)
    a3 = ex / jnp.sum(ex, axis=1, keepdims=True)
    aexp = jnp.dot(a3.reshape(R, H).astype(bf16), exp_ref[...],
                   preferred_element_type=jnp.float32).reshape(BN_, L, H * A)
    v3 = (jnp.dot(g2, wvn[...], preferred_element_type=jnp.float32)
          + jnp.dot(eo2, wve[...], preferred_element_type=jnp.float32)
          ).reshape(BN_, L, H * A) + vs[:, None, :] + bv[0, :]
    w3 = aexp * v3
    w_ref[...] = w3.astype(jnp.bfloat16)
    s = jnp.sum(w3, axis=(0, 1))
    ss = jnp.sum(w3 * w3, axis=(0, 1))
    st = jnp.concatenate([s[None, :], ss[None, :]], axis=0)

    @pl.when(i == 0)
    def _():
        acc_ref[...] = jnp.zeros_like(acc_ref)

    acc_ref[...] += st


def _t4_body(w_ref, acc3, cat_ref, acc_ref):
    i = pl.program_id(0)
    m = acc3[0, :] / NL
    rs = lax.rsqrt(acc3[1, :] / NL - m * m + EPS)
    hv = _softplus((w_ref[...].astype(jnp.float32) - m) * rs)
    cat = jnp.sum(hv, axis=1)
    cat_ref[...] = cat
    s = jnp.sum(cat, axis=0)
    ss = jnp.sum(cat * cat, axis=0)
    st = jnp.concatenate([s[None, :], ss[None, :]], axis=0)

    @pl.when(i == 0)
    def _():
        acc_ref[...] = jnp.zeros_like(acc_ref)

    acc_ref[...] += st


def _t5_body(n_ref, cat_ref, acc4, out_ref):
    m = acc4[0, :] / N
    rs = lax.rsqrt(acc4[1, :] / N - m * m + EPS)
    out_ref[...] = n_ref[...] + (cat_ref[...] - m) * rs


def _full(shape):
    nd = len(shape)
    return pl.BlockSpec(shape, lambda i: (0,) * nd)


_SEQ = pltpu.CompilerParams(dimension_semantics=("arbitrary",))


def _tc_pipeline(node_features, gathered3, edge_features, edge_W, edge_b,
                 att_W1, att_b1, att_W2, att_b2, val_W, val_b):
    f32 = jnp.float32
    bf16 = jnp.bfloat16
    wes = edge_W[:D].astype(bf16)
    wen = edge_W[D:2 * D].astype(bf16)
    wee = edge_W[2 * D:].astype(bf16)
    w1cat = jnp.transpose(att_W1, (1, 0, 2)).reshape(2 * D + DE, H * A)
    w1s = w1cat[:D].astype(bf16)
    w1n = w1cat[D:2 * D].astype(bf16)
    w1e = w1cat[2 * D:].astype(bf16)
    wvcat = jnp.transpose(val_W, (1, 0, 2)).reshape(2 * D + DE, H * A)
    wvs = wvcat[:D].astype(bf16)
    wvn = wvcat[D:2 * D].astype(bf16)
    wve = wvcat[2 * D:].astype(bf16)
    b1 = att_b1.reshape(1, H * A)
    bv = val_b.reshape(1, H * A)
    w2b = jax.scipy.linalg.block_diag(
        *[att_W2[hh] for hh in range(H)]).astype(bf16)
    b2 = att_b2.reshape(1, H)
    expander = jnp.repeat(jnp.eye(H, dtype=bf16), A, axis=1)
    eb = edge_b.reshape(1, DE)

    nspec = pl.BlockSpec((BN_, D), lambda i: (i, 0))
    gspec = pl.BlockSpec((BN_, L, D), lambda i: (i, 0, 0))
    espec = pl.BlockSpec((BN_, L, DE), lambda i: (i, 0, 0))

    acc1 = pl.pallas_call(
        _t1_body,
        grid=(NBLK,),
        in_specs=[nspec, gspec, espec, _full((D, DE)), _full((D, DE)),
                  _full((DE, DE)), _full((1, DE))],
        compiler_params=_SEQ,
        out_specs=_full((2, DE)),
        out_shape=jax.ShapeDtypeStruct((2, DE), f32),
    )(node_features, gathered3, edge_features, wes, wen, wee, eb)

    edge_out, weighted, acc3 = pl.pallas_call(
        _t3_body,
        grid=(NBLK,),
        in_specs=[nspec, gspec, espec, _full((2, DE)),
                  _full((D, DE)), _full((D, DE)), _full((DE, DE)),
                  _full((1, DE)),
                  _full((D, H * A)), _full((D, H * A)), _full((DE, H * A)),
                  _full((1, H * A)),
                  _full((D, H * A)), _full((D, H * A)), _full((DE, H * A)),
                  _full((1, H * A)),
                  _full((H * A, H)), _full((1, H)), _full((H, H * A))],
        out_specs=[espec, gspec, _full((2, H * A))],
        out_shape=[jax.ShapeDtypeStruct((N, L, DE), f32),
                   jax.ShapeDtypeStruct((N, L, H * A), jnp.bfloat16),
                   jax.ShapeDtypeStruct((2, H * A), f32)],
        compiler_params=_SEQ,
    )(node_features, gathered3, edge_features, acc1, wes, wen, wee, eb,
      w1s, w1n, w1e, b1, wvs, wvn, wve, bv, w2b, b2, expander)

    cat, acc4 = pl.pallas_call(
        _t4_body,
        grid=(NBLK,),
        in_specs=[gspec, _full((2, H * A))],
        out_specs=[nspec, _full((2, H * A))],
        out_shape=[jax.ShapeDtypeStruct((N, H * A), f32),
                   jax.ShapeDtypeStruct((2, H * A), f32)],
        compiler_params=_SEQ,
    )(weighted, acc3)

    n5spec = pl.BlockSpec((1000, D), lambda i: (i, 0))
    atom_out = pl.pallas_call(
        _t5_body,
        grid=(10,),
        in_specs=[n5spec, n5spec, _full((2, H * A))],
        out_specs=n5spec,
        out_shape=jax.ShapeDtypeStruct((N, D), f32),
        compiler_params=_SEQ,
    )(node_features, cat, acc4)

    return atom_out, edge_out


def kernel(node_features, edge_features, neighbor_indices, neighbor_masks,
           h, c, edge_W, edge_b, att_W1, att_b1, att_W2, att_b2, val_W,
           val_b):
    idx2d = neighbor_indices.astype(jnp.int32).reshape(NW, NCH, GC)
    gathered3 = _sc_gather(node_features, idx2d).reshape(N, L, D)
    atom_out, edge_out = _tc_pipeline(
        node_features, gathered3, edge_features, edge_W, edge_b,
        att_W1, att_b1, att_W2, att_b2, val_W, val_b)
    return (atom_out, edge_out, h, c)


# speedup vs baseline: 4.4195x; 1.0902x over previous
import functools

import jax
import jax.numpy as jnp
from jax import lax
from jax.experimental import pallas as pl
from jax.experimental.pallas import tpu as pltpu
from jax.experimental.pallas import tpu_sc as plsc

N, L, D, DE, H, A = 10000, 32, 128, 16, 8, 16
NL = N * L
BN_ = 80
NBLK = N // BN_
R = BN_ * L
BN2 = 128
NBLK2 = (N + BN2 - 1) // BN2
R2 = BN2 * L
NW = 32
PER_W = NL // NW
GC = 80
NCH = PER_W // GC
EPS = 1e-5


def _softplus(x):
    return jnp.maximum(x, 0.0) + jnp.log(1.0 + jnp.exp(-jnp.abs(x)))


GD = D // 2


def _gather_kernel(table_hbm, idx_hbm, out_hbm, idx_v, rows_v, sem):
    wid = lax.axis_index("s") * 2 + lax.axis_index("c")
    base = wid * PER_W
    pltpu.sync_copy(idx_hbm.at[wid], idx_v)

    def body(j, carry):
        pltpu.async_copy(table_hbm.at[idx_v.at[j]], rows_v, sem).wait()
        pltpu.sync_copy(rows_v, out_hbm.at[pl.ds(base + j * GC, GC)])
        return carry

    lax.fori_loop(0, NCH, body, 0)


def _sc_gather(table, idx2d):
    mesh = plsc.VectorSubcoreMesh(core_axis_name="c", subcore_axis_name="s")
    return pl.kernel(
        _gather_kernel,
        out_type=jax.ShapeDtypeStruct((NL, D), jnp.float32),
        mesh=mesh,
        scratch_types=[
            pltpu.VMEM((NCH, GC), jnp.int32),
            pltpu.VMEM((GC, D), jnp.float32),
            pltpu.SemaphoreType.DMA,
        ],
    )(table, idx2d)


def _node_mask(i, shape, axis):
    n0 = lax.broadcasted_iota(jnp.int32, shape, axis) + i * BN2
    return n0 < N


def _t1_body(n_ref, g_ref, e_ref, wes, wen, wee, b_ref, acc_ref):
    i = pl.program_id(0)
    bf16 = jnp.bfloat16
    g2 = g_ref[...].reshape(R2, D).astype(bf16)
    e2 = jnp.transpose(e_ref[...], (2, 0, 1)).reshape(R2, DE).astype(bf16)
    zs = jnp.dot(n_ref[...].astype(bf16), wes[...],
                 preferred_element_type=jnp.float32)
    zn = jnp.dot(g2, wen[...], preferred_element_type=jnp.float32)
    ze = jnp.dot(e2, wee[...], preferred_element_type=jnp.float32)
    e3 = zs[:, None, :] + (zn + ze).reshape(BN2, L, DE) + b_ref[0, :]
    e3 = jnp.where(_node_mask(i, (BN2, 1, 1), 0), e3, 0.0)
    s = jnp.sum(e3, axis=(0, 1))
    ss = jnp.sum(e3 * e3, axis=(0, 1))
    st = jnp.concatenate([s[None, :], ss[None, :]], axis=0)

    @pl.when(i == 0)
    def _():
        acc_ref[...] = jnp.zeros_like(acc_ref)

    acc_ref[...] += st


def _t3_body(n_ref, g_ref, e_ref, acc1, wes, wen, wee, eb, w1s, w1n, w1e, b1,
             wvs, wvn, wve, bv, w2b, b2, exp_ref, eo_ref, w_ref, acc_ref):
    i = pl.program_id(0)
    bf16 = jnp.bfloat16
    g2 = g_ref[...].reshape(R2, D).astype(bf16)
    nblk = n_ref[...].astype(bf16)
    e3d = jnp.transpose(e_ref[...], (2, 0, 1))
    e2 = e3d.reshape(R2, DE).astype(bf16)
    zs = jnp.dot(nblk, wes[...], preferred_element_type=jnp.float32)
    zn = jnp.dot(g2, wen[...], preferred_element_type=jnp.float32)
    ze = jnp.dot(e2, wee[...], preferred_element_type=jnp.float32)
    ep3 = zs[:, None, :] + (zn + ze).reshape(BN2, L, DE) + eb[0, :]
    m1 = acc1[0, :] / NL
    rs1 = lax.rsqrt(acc1[1, :] / NL - m1 * m1 + EPS)
    eo3 = _softplus(e3d + (ep3 - m1) * rs1)
    eo_ref[...] = jnp.transpose(eo3, (1, 2, 0))
    eo2 = eo3.reshape(R2, DE).astype(bf16)
    ps = jnp.dot(nblk, w1s[...], preferred_element_type=jnp.float32)
    vs = jnp.dot(nblk, wvs[...], preferred_element_type=jnp.float32)
    hid3 = _softplus(
        (jnp.dot(g2, w1n[...], preferred_element_type=jnp.float32)
         + jnp.dot(eo2, w1e[...], preferred_element_type=jnp.float32)
         ).reshape(BN2, L, H * A) + ps[:, None, :] + b1[0, :])
    lg = (jnp.dot(hid3.reshape(R2, H * A).astype(bf16), w2b[...],
                  preferred_element_type=jnp.float32) + b2[0, :]
          ).reshape(BN2, L, H)
    mx = jnp.max(lg, axis=1, keepdims=True)
    ex = jnp.exp(lg - mx)
    a3 = ex / jnp.sum(ex, axis=1, keepdims=True)
    aexp = jnp.dot(a3.reshape(R2, H).astype(bf16), exp_ref[...],
                   preferred_element_type=jnp.float32).reshape(BN2, L, H * A)
    v3 = (jnp.dot(g2, wvn[...], preferred_element_type=jnp.float32)
          + jnp.dot(eo2, wve[...], preferred_element_type=jnp.float32)
          ).reshape(BN2, L, H * A) + vs[:, None, :] + bv[0, :]
    w3 = jnp.where(_node_mask(i, (BN2, 1, 1), 0), aexp * v3, 0.0)
    w_ref[...] = w3.astype(jnp.bfloat16)
    s = jnp.sum(w3, axis=(0, 1))
    ss = jnp.sum(w3 * w3, axis=(0, 1))
    st = jnp.concatenate([s[None, :], ss[None, :]], axis=0)

    @pl.when(i == 0)
    def _():
        acc_ref[...] = jnp.zeros_like(acc_ref)

    acc_ref[...] += st


def _t4_body(w_ref, acc3, cat_ref, acc_ref):
    i = pl.program_id(0)
    m = acc3[0, :] / NL
    rs = lax.rsqrt(acc3[1, :] / NL - m * m + EPS)
    hv = _softplus((w_ref[...].astype(jnp.float32) - m) * rs)
    cat = jnp.sum(hv, axis=1)
    cat_ref[...] = cat
    s = jnp.sum(cat, axis=0)
    ss = jnp.sum(cat * cat, axis=0)
    st = jnp.concatenate([s[None, :], ss[None, :]], axis=0)

    @pl.when(i == 0)
    def _():
        acc_ref[...] = jnp.zeros_like(acc_ref)

    acc_ref[...] += st


def _t5_body(n_ref, cat_ref, acc4, out_ref):
    m = acc4[0, :] / N
    rs = lax.rsqrt(acc4[1, :] / N - m * m + EPS)
    out_ref[...] = n_ref[...] + (cat_ref[...] - m) * rs


def _full(shape):
    nd = len(shape)
    return pl.BlockSpec(shape, lambda i: (0,) * nd)


_SEQ = pltpu.CompilerParams(dimension_semantics=("arbitrary",))


def _tc_pipeline(node_features, gathered3, edge_features, edge_W, edge_b,
                 att_W1, att_b1, att_W2, att_b2, val_W, val_b):
    f32 = jnp.float32
    bf16 = jnp.bfloat16
    wes = edge_W[:D].astype(bf16)
    wen = edge_W[D:2 * D].astype(bf16)
    wee = edge_W[2 * D:].astype(bf16)
    w1cat = jnp.transpose(att_W1, (1, 0, 2)).reshape(2 * D + DE, H * A)
    w1s = w1cat[:D].astype(bf16)
    w1n = w1cat[D:2 * D].astype(bf16)
    w1e = w1cat[2 * D:].astype(bf16)
    wvcat = jnp.transpose(val_W, (1, 0, 2)).reshape(2 * D + DE, H * A)
    wvs = wvcat[:D].astype(bf16)
    wvn = wvcat[D:2 * D].astype(bf16)
    wve = wvcat[2 * D:].astype(bf16)
    b1 = att_b1.reshape(1, H * A)
    bv = val_b.reshape(1, H * A)
    w2b = jax.scipy.linalg.block_diag(
        *[att_W2[hh] for hh in range(H)]).astype(bf16)
    b2 = att_b2.reshape(1, H)
    expander = jnp.repeat(jnp.eye(H, dtype=bf16), A, axis=1)
    eb = edge_b.reshape(1, DE)

    nspec = pl.BlockSpec((BN_, D), lambda i: (i, 0))
    gspec = pl.BlockSpec((BN_, L, D), lambda i: (i, 0, 0))
    nspec2 = pl.BlockSpec((BN2, D), lambda i: (i, 0))
    gspec2 = pl.BlockSpec((BN2, L, D), lambda i: (i, 0, 0))
    espec_t = pl.BlockSpec((L, DE, BN2), lambda i: (0, 0, i))
    ef_t = jnp.transpose(edge_features, (1, 2, 0))

    acc1 = pl.pallas_call(
        _t1_body,
        grid=(NBLK2,),
        in_specs=[nspec2, gspec2, espec_t, _full((D, DE)), _full((D, DE)),
                  _full((DE, DE)), _full((1, DE))],
        compiler_params=_SEQ,
        out_specs=_full((2, DE)),
        out_shape=jax.ShapeDtypeStruct((2, DE), f32),
    )(node_features, gathered3, ef_t, wes, wen, wee, eb)

    edge_out_t, weighted, acc3 = pl.pallas_call(
        _t3_body,
        grid=(NBLK2,),
        in_specs=[nspec2, gspec2, espec_t, _full((2, DE)),
                  _full((D, DE)), _full((D, DE)), _full((DE, DE)),
                  _full((1, DE)),
                  _full((D, H * A)), _full((D, H * A)), _full((DE, H * A)),
                  _full((1, H * A)),
                  _full((D, H * A)), _full((D, H * A)), _full((DE, H * A)),
                  _full((1, H * A)),
                  _full((H * A, H)), _full((1, H)), _full((H, H * A))],
        out_specs=[espec_t, gspec2, _full((2, H * A))],
        out_shape=[jax.ShapeDtypeStruct((L, DE, N), f32),
                   jax.ShapeDtypeStruct((N, L, H * A), jnp.bfloat16),
                   jax.ShapeDtypeStruct((2, H * A), f32)],
        compiler_params=_SEQ,
    )(node_features, gathered3, ef_t, acc1, wes, wen, wee, eb,
      w1s, w1n, w1e, b1, wvs, wvn, wve, bv, w2b, b2, expander)
    edge_out = jnp.transpose(edge_out_t, (2, 0, 1))

    cat, acc4 = pl.pallas_call(
        _t4_body,
        grid=(NBLK,),
        in_specs=[gspec, _full((2, H * A))],
        out_specs=[nspec, _full((2, H * A))],
        out_shape=[jax.ShapeDtypeStruct((N, H * A), f32),
                   jax.ShapeDtypeStruct((2, H * A), f32)],
        compiler_params=_SEQ,
    )(weighted, acc3)

    n5spec = pl.BlockSpec((1000, D), lambda i: (i, 0))
    atom_out = pl.pallas_call(
        _t5_body,
        grid=(10,),
        in_specs=[n5spec, n5spec, _full((2, H * A))],
        out_specs=n5spec,
        out_shape=jax.ShapeDtypeStruct((N, D), f32),
        compiler_params=_SEQ,
    )(node_features, cat, acc4)

    return atom_out, edge_out


def kernel(node_features, edge_features, neighbor_indices, neighbor_masks,
           h, c, edge_W, edge_b, att_W1, att_b1, att_W2, att_b2, val_W,
           val_b):
    idx2d = neighbor_indices.astype(jnp.int32).reshape(NW, NCH, GC)
    gathered3 = _sc_gather(node_features, idx2d).reshape(N, L, D)
    atom_out, edge_out = _tc_pipeline(
        node_features, gathered3, edge_features, edge_W, edge_b,
        att_W1, att_b1, att_W2, att_b2, val_W, val_b)
    return (atom_out, edge_out, h, c)


# speedup vs baseline: 4.5882x; 1.0382x over previous
import functools

import jax
import jax.numpy as jnp
from jax import lax
from jax.experimental import pallas as pl
from jax.experimental.pallas import tpu as pltpu
from jax.experimental.pallas import tpu_sc as plsc

N, L, D, DE, H, A = 10000, 32, 128, 16, 8, 16
NL = N * L
BN_ = 80
NBLK = N // BN_
R = BN_ * L
BN2 = 128
NBLK2 = (N + BN2 - 1) // BN2
R2 = BN2 * L
NW = 32
PER_W = NL // NW
GC = 80
NCH = PER_W // GC
EPS = 1e-5


def _softplus(x):
    return jnp.maximum(x, 0.0) + jnp.log(1.0 + jnp.exp(-jnp.abs(x)))


GD = D // 2


def _gather_kernel(table_hbm, idx_hbm, out_hbm, idx_v, rows_v, sem):
    wid = lax.axis_index("s") * 2 + lax.axis_index("c")
    base = wid * PER_W
    pltpu.sync_copy(idx_hbm.at[wid], idx_v)

    def body(j, carry):
        pltpu.async_copy(table_hbm.at[idx_v.at[j]], rows_v, sem).wait()
        pltpu.sync_copy(rows_v, out_hbm.at[pl.ds(base + j * GC, GC)])
        return carry

    lax.fori_loop(0, NCH, body, 0)


def _sc_gather(table, idx2d):
    mesh = plsc.VectorSubcoreMesh(core_axis_name="c", subcore_axis_name="s")
    return pl.kernel(
        _gather_kernel,
        out_type=jax.ShapeDtypeStruct((NL, D), jnp.float32),
        mesh=mesh,
        scratch_types=[
            pltpu.VMEM((NCH, GC), jnp.int32),
            pltpu.VMEM((GC, D), jnp.float32),
            pltpu.SemaphoreType.DMA,
        ],
    )(table, idx2d)


def _node_mask(i, shape, axis):
    n0 = lax.broadcasted_iota(jnp.int32, shape, axis) + i * BN2
    return n0 < N


def _t1_body(n_ref, g_ref, e_ref, wes, wen, wee, b_ref, acc_ref):
    i = pl.program_id(0)
    bf16 = jnp.bfloat16
    g2 = g_ref[...].reshape(R2, D).astype(bf16)
    e2 = jnp.transpose(e_ref[...].astype(bf16), (2, 0, 1)).reshape(R2, DE)
    zs = jnp.dot(n_ref[...].astype(bf16), wes[...],
                 preferred_element_type=jnp.float32)
    zn = jnp.dot(g2, wen[...], preferred_element_type=jnp.float32)
    ze = jnp.dot(e2, wee[...], preferred_element_type=jnp.float32)
    e3 = zs[:, None, :] + (zn + ze).reshape(BN2, L, DE) + b_ref[0, :]
    e3 = jnp.where(_node_mask(i, (BN2, 1, 1), 0), e3, 0.0)
    s = jnp.sum(e3, axis=(0, 1))
    ss = jnp.sum(e3 * e3, axis=(0, 1))
    st = jnp.concatenate([s[None, :], ss[None, :]], axis=0)

    @pl.when(i == 0)
    def _():
        acc_ref[...] = jnp.zeros_like(acc_ref)

    acc_ref[...] += st


def _t3_body(n_ref, g_ref, e_ref, acc1, wes, wen, wee, eb, w1s, w1n, w1e, b1,
             wvs, wvn, wve, bv, w2b, b2, exp_ref, eo_ref, w_ref, acc_ref):
    i = pl.program_id(0)
    bf16 = jnp.bfloat16
    g2 = g_ref[...].reshape(R2, D).astype(bf16)
    nblk = n_ref[...].astype(bf16)
    e3d = jnp.transpose(e_ref[...].astype(bf16), (2, 0, 1))
    e2 = e3d.reshape(R2, DE)
    zs = jnp.dot(nblk, wes[...], preferred_element_type=jnp.float32)
    zn = jnp.dot(g2, wen[...], preferred_element_type=jnp.float32)
    ze = jnp.dot(e2, wee[...], preferred_element_type=jnp.float32)
    ep3 = zs[:, None, :] + (zn + ze).reshape(BN2, L, DE) + eb[0, :]
    m1 = acc1[0, :] / NL
    rs1 = lax.rsqrt(acc1[1, :] / NL - m1 * m1 + EPS)
    eo3 = _softplus(e3d.astype(jnp.float32) + (ep3 - m1) * rs1)
    eo_ref[...] = jnp.transpose(eo3.astype(bf16), (1, 2, 0)
                                ).astype(jnp.float32)
    eo2 = eo3.reshape(R2, DE).astype(bf16)
    ps = jnp.dot(nblk, w1s[...], preferred_element_type=jnp.float32)
    vs = jnp.dot(nblk, wvs[...], preferred_element_type=jnp.float32)
    hid3 = _softplus(
        (jnp.dot(g2, w1n[...], preferred_element_type=jnp.float32)
         + jnp.dot(eo2, w1e[...], preferred_element_type=jnp.float32)
         ).reshape(BN2, L, H * A) + ps[:, None, :] + b1[0, :])
    lg = (jnp.dot(hid3.reshape(R2, H * A).astype(bf16), w2b[...],
                  preferred_element_type=jnp.float32) + b2[0, :]
          ).reshape(BN2, L, H)
    mx = jnp.max(lg, axis=1, keepdims=True)
    ex = jnp.exp(lg - mx)
    a3 = ex / jnp.sum(ex, axis=1, keepdims=True)
    aexp = jnp.dot(a3.reshape(R2, H).astype(bf16), exp_ref[...],
                   preferred_element_type=jnp.float32).reshape(BN2, L, H * A)
    v3 = (jnp.dot(g2, wvn[...], preferred_element_type=jnp.float32)
          + jnp.dot(eo2, wve[...], preferred_element_type=jnp.float32)
          ).reshape(BN2, L, H * A) + vs[:, None, :] + bv[0, :]
    w3 = jnp.where(_node_mask(i, (BN2, 1, 1), 0), aexp * v3, 0.0)
    w_ref[...] = w3.astype(jnp.bfloat16)
    s = jnp.sum(w3, axis=(0, 1))
    ss = jnp.sum(w3 * w3, axis=(0, 1))
    st = jnp.concatenate([s[None, :], ss[None, :]], axis=0)

    @pl.when(i == 0)
    def _():
        acc_ref[...] = jnp.zeros_like(acc_ref)

    acc_ref[...] += st


def _t4_body(w_ref, acc3, cat_ref, acc_ref):
    i = pl.program_id(0)
    m = acc3[0, :] / NL
    rs = lax.rsqrt(acc3[1, :] / NL - m * m + EPS)
    hv = _softplus((w_ref[...].astype(jnp.float32) - m) * rs)
    cat = jnp.sum(hv, axis=1)
    cat_ref[...] = cat
    s = jnp.sum(cat, axis=0)
    ss = jnp.sum(cat * cat, axis=0)
    st = jnp.concatenate([s[None, :], ss[None, :]], axis=0)

    @pl.when(i == 0)
    def _():
        acc_ref[...] = jnp.zeros_like(acc_ref)

    acc_ref[...] += st


def _t5_body(n_ref, cat_ref, acc4, out_ref):
    m = acc4[0, :] / N
    rs = lax.rsqrt(acc4[1, :] / N - m * m + EPS)
    out_ref[...] = n_ref[...] + (cat_ref[...] - m) * rs


def _full(shape):
    nd = len(shape)
    return pl.BlockSpec(shape, lambda i: (0,) * nd)


_SEQ = pltpu.CompilerParams(dimension_semantics=("arbitrary",))


def _tc_pipeline(node_features, gathered3, edge_features, edge_W, edge_b,
                 att_W1, att_b1, att_W2, att_b2, val_W, val_b):
    f32 = jnp.float32
    bf16 = jnp.bfloat16
    wes = edge_W[:D].astype(bf16)
    wen = edge_W[D:2 * D].astype(bf16)
    wee = edge_W[2 * D:].astype(bf16)
    w1cat = jnp.transpose(att_W1, (1, 0, 2)).reshape(2 * D + DE, H * A)
    w1s = w1cat[:D].astype(bf16)
    w1n = w1cat[D:2 * D].astype(bf16)
    w1e = w1cat[2 * D:].astype(bf16)
    wvcat = jnp.transpose(val_W, (1, 0, 2)).reshape(2 * D + DE, H * A)
    wvs = wvcat[:D].astype(bf16)
    wvn = wvcat[D:2 * D].astype(bf16)
    wve = wvcat[2 * D:].astype(bf16)
    b1 = att_b1.reshape(1, H * A)
    bv = val_b.reshape(1, H * A)
    w2b = jax.scipy.linalg.block_diag(
        *[att_W2[hh] for hh in range(H)]).astype(bf16)
    b2 = att_b2.reshape(1, H)
    expander = jnp.repeat(jnp.eye(H, dtype=bf16), A, axis=1)
    eb = edge_b.reshape(1, DE)

    nspec = pl.BlockSpec((BN_, D), lambda i: (i, 0))
    gspec = pl.BlockSpec((BN_, L, D), lambda i: (i, 0, 0))
    nspec2 = pl.BlockSpec((BN2, D), lambda i: (i, 0))
    gspec2 = pl.BlockSpec((BN2, L, D), lambda i: (i, 0, 0))
    espec_t = pl.BlockSpec((L, DE, BN2), lambda i: (0, 0, i))
    ef_t = jnp.transpose(edge_features, (1, 2, 0))

    acc1 = pl.pallas_call(
        _t1_body,
        grid=(NBLK2,),
        in_specs=[nspec2, gspec2, espec_t, _full((D, DE)), _full((D, DE)),
                  _full((DE, DE)), _full((1, DE))],
        compiler_params=_SEQ,
        out_specs=_full((2, DE)),
        out_shape=jax.ShapeDtypeStruct((2, DE), f32),
    )(node_features, gathered3, ef_t, wes, wen, wee, eb)

    edge_out_t, weighted, acc3 = pl.pallas_call(
        _t3_body,
        grid=(NBLK2,),
        in_specs=[nspec2, gspec2, espec_t, _full((2, DE)),
                  _full((D, DE)), _full((D, DE)), _full((DE, DE)),
                  _full((1, DE)),
                  _full((D, H * A)), _full((D, H * A)), _full((DE, H * A)),
                  _full((1, H * A)),
                  _full((D, H * A)), _full((D, H * A)), _full((DE, H * A)),
                  _full((1, H * A)),
                  _full((H * A, H)), _full((1, H)), _full((H, H * A))],
        out_specs=[espec_t, gspec2, _full((2, H * A))],
        out_shape=[jax.ShapeDtypeStruct((L, DE, N), f32),
                   jax.ShapeDtypeStruct((N, L, H * A), jnp.bfloat16),
                   jax.ShapeDtypeStruct((2, H * A), f32)],
        compiler_params=_SEQ,
    )(node_features, gathered3, ef_t, acc1, wes, wen, wee, eb,
      w1s, w1n, w1e, b1, wvs, wvn, wve, bv, w2b, b2, expander)
    edge_out = jnp.transpose(edge_out_t, (2, 0, 1))

    cat, acc4 = pl.pallas_call(
        _t4_body,
        grid=(NBLK,),
        in_specs=[gspec, _full((2, H * A))],
        out_specs=[nspec, _full((2, H * A))],
        out_shape=[jax.ShapeDtypeStruct((N, H * A), f32),
                   jax.ShapeDtypeStruct((2, H * A), f32)],
        compiler_params=_SEQ,
    )(weighted, acc3)

    n5spec = pl.BlockSpec((1000, D), lambda i: (i, 0))
    atom_out = pl.pallas_call(
        _t5_body,
        grid=(10,),
        in_specs=[n5spec, n5spec, _full((2, H * A))],
        out_specs=n5spec,
        out_shape=jax.ShapeDtypeStruct((N, D), f32),
        compiler_params=_SEQ,
    )(node_features, cat, acc4)

    return atom_out, edge_out


def kernel(node_features, edge_features, neighbor_indices, neighbor_masks,
           h, c, edge_W, edge_b, att_W1, att_b1, att_W2, att_b2, val_W,
           val_b):
    idx2d = neighbor_indices.astype(jnp.int32).reshape(NW, NCH, GC)
    gathered3 = _sc_gather(node_features, idx2d).reshape(N, L, D)
    atom_out, edge_out = _tc_pipeline(
        node_features, gathered3, edge_features, edge_W, edge_b,
        att_W1, att_b1, att_W2, att_b2, val_W, val_b)
    return (atom_out, edge_out, h, c)


# speedup vs baseline: 5.0402x; 1.0985x over previous
import functools

import jax
import jax.numpy as jnp
from jax import lax
from jax.experimental import pallas as pl
from jax.experimental.pallas import tpu as pltpu
from jax.experimental.pallas import tpu_sc as plsc

N, L, D, DE, H, A = 10000, 32, 128, 16, 8, 16
NL = N * L
BN_ = 80
NBLK = N // BN_
R = BN_ * L
BN2 = 128
NBLK2 = (N + BN2 - 1) // BN2
R2 = BN2 * L
NW = 32
PER_W = NL // NW
GC = 80
NCH = PER_W // GC
EPS = 1e-5


def _softplus(x):
    return jnp.maximum(x, 0.0) + jnp.log(1.0 + jnp.exp(-jnp.abs(x)))


GD = D // 2


def _gather_kernel(table_hbm, idx_hbm, out_hbm, idx_v, rows0, rows1,
                   sem0, sem1):
    wid = lax.axis_index("s") * 2 + lax.axis_index("c")
    base = wid * PER_W
    pltpu.sync_copy(idx_hbm.at[wid], idx_v)
    pltpu.async_copy(table_hbm.at[idx_v.at[0]], rows0, sem0)

    def body(p, carry):
        j0 = 2 * p
        j1 = j0 + 1
        pltpu.async_copy(table_hbm.at[idx_v.at[j1]], rows1, sem1)
        pltpu.make_async_copy(table_hbm.at[idx_v.at[j0]], rows0, sem0).wait()
        pltpu.sync_copy(rows0, out_hbm.at[pl.ds(base + j0 * GC, GC)])
        pltpu.async_copy(table_hbm.at[idx_v.at[j0 + 2]], rows0, sem0)
        pltpu.make_async_copy(table_hbm.at[idx_v.at[j1]], rows1, sem1).wait()
        pltpu.sync_copy(rows1, out_hbm.at[pl.ds(base + j1 * GC, GC)])
        return carry

    lax.fori_loop(0, (NCH - 1) // 2, body, 0)
    jl = NCH - 1
    pltpu.make_async_copy(table_hbm.at[idx_v.at[jl]], rows0, sem0).wait()
    pltpu.sync_copy(rows0, out_hbm.at[pl.ds(base + jl * GC, GC)])


def _sc_gather(table, idx2d):
    mesh = plsc.VectorSubcoreMesh(core_axis_name="c", subcore_axis_name="s")
    return pl.kernel(
        _gather_kernel,
        out_type=jax.ShapeDtypeStruct((NL, D), jnp.float32),
        mesh=mesh,
        scratch_types=[
            pltpu.VMEM((NCH, GC), jnp.int32),
            pltpu.VMEM((GC, D), jnp.float32),
            pltpu.VMEM((GC, D), jnp.float32),
            pltpu.SemaphoreType.DMA,
            pltpu.SemaphoreType.DMA,
        ],
    )(table, idx2d)


def _node_mask(i, shape, axis):
    n0 = lax.broadcasted_iota(jnp.int32, shape, axis) + i * BN2
    return n0 < N


def _t1_body(n_ref, g_ref, e_ref, wes, wen, wee, b_ref, acc_ref):
    i = pl.program_id(0)
    bf16 = jnp.bfloat16
    g2 = g_ref[...].reshape(R2, D).astype(bf16)
    e2 = jnp.transpose(e_ref[...].astype(bf16), (2, 0, 1)).reshape(R2, DE)
    zs = jnp.dot(n_ref[...].astype(bf16), wes[...],
                 preferred_element_type=jnp.float32)
    zn = jnp.dot(g2, wen[...], preferred_element_type=jnp.float32)
    ze = jnp.dot(e2, wee[...], preferred_element_type=jnp.float32)
    e3 = zs[:, None, :] + (zn + ze).reshape(BN2, L, DE) + b_ref[0, :]
    e3 = jnp.where(_node_mask(i, (BN2, 1, 1), 0), e3, 0.0)
    s = jnp.sum(e3, axis=(0, 1))
    ss = jnp.sum(e3 * e3, axis=(0, 1))
    st = jnp.concatenate([s[None, :], ss[None, :]], axis=0)

    @pl.when(i == 0)
    def _():
        acc_ref[...] = jnp.zeros_like(acc_ref)

    acc_ref[...] += st


def _t3_body(n_ref, g_ref, e_ref, acc1, wes, wen, wee, eb, w1s, w1n, w1e, b1,
             wvs, wvn, wve, bv, w2b, b2, exp_ref, eo_ref, w_ref, acc_ref):
    i = pl.program_id(0)
    bf16 = jnp.bfloat16
    g2 = g_ref[...].reshape(R2, D).astype(bf16)
    nblk = n_ref[...].astype(bf16)
    e3d = jnp.transpose(e_ref[...].astype(bf16), (2, 0, 1))
    e2 = e3d.reshape(R2, DE)
    zs = jnp.dot(nblk, wes[...], preferred_element_type=jnp.float32)
    zn = jnp.dot(g2, wen[...], preferred_element_type=jnp.float32)
    ze = jnp.dot(e2, wee[...], preferred_element_type=jnp.float32)
    ep3 = zs[:, None, :] + (zn + ze).reshape(BN2, L, DE) + eb[0, :]
    m1 = acc1[0, :] / NL
    rs1 = lax.rsqrt(acc1[1, :] / NL - m1 * m1 + EPS)
    eo3 = _softplus(e3d.astype(jnp.float32) + (ep3 - m1) * rs1)
    eo_ref[...] = jnp.transpose(eo3.astype(bf16), (1, 2, 0)
                                ).astype(jnp.float32)
    eo2 = eo3.reshape(R2, DE).astype(bf16)
    ps = jnp.dot(nblk, w1s[...], preferred_element_type=jnp.float32)
    vs = jnp.dot(nblk, wvs[...], preferred_element_type=jnp.float32)
    hid3 = _softplus(
        (jnp.dot(g2, w1n[...], preferred_element_type=jnp.float32)
         + jnp.dot(eo2, w1e[...], preferred_element_type=jnp.float32)
         ).reshape(BN2, L, H * A) + ps[:, None, :] + b1[0, :])
    lg = (jnp.dot(hid3.reshape(R2, H * A).astype(bf16), w2b[...],
                  preferred_element_type=jnp.float32) + b2[0, :]
          ).reshape(BN2, L, H)
    mx = jnp.max(lg, axis=1, keepdims=True)
    ex = jnp.exp(lg - mx)
    a3 = ex / jnp.sum(ex, axis=1, keepdims=True)
    aexp = jnp.dot(a3.reshape(R2, H).astype(bf16), exp_ref[...],
                   preferred_element_type=jnp.float32).reshape(BN2, L, H * A)
    v3 = (jnp.dot(g2, wvn[...], preferred_element_type=jnp.float32)
          + jnp.dot(eo2, wve[...], preferred_element_type=jnp.float32)
          ).reshape(BN2, L, H * A) + vs[:, None, :] + bv[0, :]
    w3 = jnp.where(_node_mask(i, (BN2, 1, 1), 0), aexp * v3, 0.0)
    w_ref[...] = w3.astype(jnp.bfloat16)
    s = jnp.sum(w3, axis=(0, 1))
    ss = jnp.sum(w3 * w3, axis=(0, 1))
    st = jnp.concatenate([s[None, :], ss[None, :]], axis=0)

    @pl.when(i == 0)
    def _():
        acc_ref[...] = jnp.zeros_like(acc_ref)

    acc_ref[...] += st


def _t4_body(w_ref, acc3, cat_ref, acc_ref):
    i = pl.program_id(0)
    m = acc3[0, :] / NL
    rs = lax.rsqrt(acc3[1, :] / NL - m * m + EPS)
    hv = _softplus((w_ref[...].astype(jnp.float32) - m) * rs)
    cat = jnp.sum(hv, axis=1)
    cat_ref[...] = cat
    s = jnp.sum(cat, axis=0)
    ss = jnp.sum(cat * cat, axis=0)
    st = jnp.concatenate([s[None, :], ss[None, :]], axis=0)

    @pl.when(i == 0)
    def _():
        acc_ref[...] = jnp.zeros_like(acc_ref)

    acc_ref[...] += st


def _t5_body(n_ref, cat_ref, acc4, out_ref):
    m = acc4[0, :] / N
    rs = lax.rsqrt(acc4[1, :] / N - m * m + EPS)
    out_ref[...] = n_ref[...] + (cat_ref[...] - m) * rs


def _full(shape):
    nd = len(shape)
    return pl.BlockSpec(shape, lambda i: (0,) * nd)


_SEQ = pltpu.CompilerParams(dimension_semantics=("arbitrary",))


def _tc_pipeline(node_features, gathered3, edge_features, edge_W, edge_b,
                 att_W1, att_b1, att_W2, att_b2, val_W, val_b):
    f32 = jnp.float32
    bf16 = jnp.bfloat16
    wes = edge_W[:D].astype(bf16)
    wen = edge_W[D:2 * D].astype(bf16)
    wee = edge_W[2 * D:].astype(bf16)
    w1cat = jnp.transpose(att_W1, (1, 0, 2)).reshape(2 * D + DE, H * A)
    w1s = w1cat[:D].astype(bf16)
    w1n = w1cat[D:2 * D].astype(bf16)
    w1e = w1cat[2 * D:].astype(bf16)
    wvcat = jnp.transpose(val_W, (1, 0, 2)).reshape(2 * D + DE, H * A)
    wvs = wvcat[:D].astype(bf16)
    wvn = wvcat[D:2 * D].astype(bf16)
    wve = wvcat[2 * D:].astype(bf16)
    b1 = att_b1.reshape(1, H * A)
    bv = val_b.reshape(1, H * A)
    w2b = jax.scipy.linalg.block_diag(
        *[att_W2[hh] for hh in range(H)]).astype(bf16)
    b2 = att_b2.reshape(1, H)
    expander = jnp.repeat(jnp.eye(H, dtype=bf16), A, axis=1)
    eb = edge_b.reshape(1, DE)

    nspec = pl.BlockSpec((BN_, D), lambda i: (i, 0))
    gspec = pl.BlockSpec((BN_, L, D), lambda i: (i, 0, 0))
    nspec2 = pl.BlockSpec((BN2, D), lambda i: (i, 0))
    gspec2 = pl.BlockSpec((BN2, L, D), lambda i: (i, 0, 0))
    espec_t = pl.BlockSpec((L, DE, BN2), lambda i: (0, 0, i))
    ef_t = jnp.transpose(edge_features, (1, 2, 0))

    acc1 = pl.pallas_call(
        _t1_body,
        grid=(NBLK2,),
        in_specs=[nspec2, gspec2, espec_t, _full((D, DE)), _full((D, DE)),
                  _full((DE, DE)), _full((1, DE))],
        compiler_params=_SEQ,
        out_specs=_full((2, DE)),
        out_shape=jax.ShapeDtypeStruct((2, DE), f32),
    )(node_features, gathered3, ef_t, wes, wen, wee, eb)

    edge_out_t, weighted, acc3 = pl.pallas_call(
        _t3_body,
        grid=(NBLK2,),
        in_specs=[nspec2, gspec2, espec_t, _full((2, DE)),
                  _full((D, DE)), _full((D, DE)), _full((DE, DE)),
                  _full((1, DE)),
                  _full((D, H * A)), _full((D, H * A)), _full((DE, H * A)),
                  _full((1, H * A)),
                  _full((D, H * A)), _full((D, H * A)), _full((DE, H * A)),
                  _full((1, H * A)),
                  _full((H * A, H)), _full((1, H)), _full((H, H * A))],
        out_specs=[espec_t, gspec2, _full((2, H * A))],
        out_shape=[jax.ShapeDtypeStruct((L, DE, N), f32),
                   jax.ShapeDtypeStruct((N, L, H * A), jnp.bfloat16),
                   jax.ShapeDtypeStruct((2, H * A), f32)],
        compiler_params=_SEQ,
    )(node_features, gathered3, ef_t, acc1, wes, wen, wee, eb,
      w1s, w1n, w1e, b1, wvs, wvn, wve, bv, w2b, b2, expander)
    edge_out = jnp.transpose(edge_out_t, (2, 0, 1))

    cat, acc4 = pl.pallas_call(
        _t4_body,
        grid=(NBLK,),
        in_specs=[gspec, _full((2, H * A))],
        out_specs=[nspec, _full((2, H * A))],
        out_shape=[jax.ShapeDtypeStruct((N, H * A), f32),
                   jax.ShapeDtypeStruct((2, H * A), f32)],
        compiler_params=_SEQ,
    )(weighted, acc3)

    n5spec = pl.BlockSpec((1000, D), lambda i: (i, 0))
    atom_out = pl.pallas_call(
        _t5_body,
        grid=(10,),
        in_specs=[n5spec, n5spec, _full((2, H * A))],
        out_specs=n5spec,
        out_shape=jax.ShapeDtypeStruct((N, D), f32),
        compiler_params=_SEQ,
    )(node_features, cat, acc4)

    return atom_out, edge_out


def kernel(node_features, edge_features, neighbor_indices, neighbor_masks,
           h, c, edge_W, edge_b, att_W1, att_b1, att_W2, att_b2, val_W,
           val_b):
    idx2d = neighbor_indices.astype(jnp.int32).reshape(NW, NCH, GC)
    gathered3 = _sc_gather(node_features, idx2d).reshape(N, L, D)
    atom_out, edge_out = _tc_pipeline(
        node_features, gathered3, edge_features, edge_W, edge_b,
        att_W1, att_b1, att_W2, att_b2, val_W, val_b)
    return (atom_out, edge_out, h, c)


# speedup vs baseline: 5.1843x; 1.0286x over previous
import functools

import jax
import jax.numpy as jnp
from jax import lax
from jax.experimental import pallas as pl
from jax.experimental.pallas import tpu as pltpu
from jax.experimental.pallas import tpu_sc as plsc

N, L, D, DE, H, A = 10000, 32, 128, 16, 8, 16
NL = N * L
BN_ = 200
NBLK = N // BN_
R = BN_ * L
BN2 = 128
NBLK2 = (N + BN2 - 1) // BN2
R2 = BN2 * L
NW = 32
PER_W = NL // NW
GC = 80
NCH = PER_W // GC
EPS = 1e-5


def _softplus(x):
    return jnp.maximum(x, 0.0) + jnp.log(1.0 + jnp.exp(-jnp.abs(x)))


GD = D // 2


def _gather_kernel(table_hbm, idx_hbm, out_hbm, idx_v, rows0, rows1,
                   sem0, sem1):
    wid = lax.axis_index("s") * 2 + lax.axis_index("c")
    base = wid * PER_W
    pltpu.sync_copy(idx_hbm.at[wid], idx_v)
    pltpu.async_copy(table_hbm.at[idx_v.at[0]], rows0, sem0)

    def body(p, carry):
        j0 = 2 * p
        j1 = j0 + 1
        pltpu.async_copy(table_hbm.at[idx_v.at[j1]], rows1, sem1)
        pltpu.make_async_copy(table_hbm.at[idx_v.at[j0]], rows0, sem0).wait()
        pltpu.sync_copy(rows0, out_hbm.at[pl.ds(base + j0 * GC, GC)])
        pltpu.async_copy(table_hbm.at[idx_v.at[j0 + 2]], rows0, sem0)
        pltpu.make_async_copy(table_hbm.at[idx_v.at[j1]], rows1, sem1).wait()
        pltpu.sync_copy(rows1, out_hbm.at[pl.ds(base + j1 * GC, GC)])
        return carry

    lax.fori_loop(0, (NCH - 1) // 2, body, 0)
    jl = NCH - 1
    pltpu.make_async_copy(table_hbm.at[idx_v.at[jl]], rows0, sem0).wait()
    pltpu.sync_copy(rows0, out_hbm.at[pl.ds(base + jl * GC, GC)])


def _sc_gather(table, idx2d):
    mesh = plsc.VectorSubcoreMesh(core_axis_name="c", subcore_axis_name="s")
    return pl.kernel(
        _gather_kernel,
        out_type=jax.ShapeDtypeStruct((NL, D), jnp.float32),
        mesh=mesh,
        scratch_types=[
            pltpu.VMEM((NCH, GC), jnp.int32),
            pltpu.VMEM((GC, D), jnp.float32),
            pltpu.VMEM((GC, D), jnp.float32),
            pltpu.SemaphoreType.DMA,
            pltpu.SemaphoreType.DMA,
        ],
    )(table, idx2d)


def _node_mask(i, shape, axis):
    n0 = lax.broadcasted_iota(jnp.int32, shape, axis) + i * BN2
    return n0 < N


def _t1_body(n_ref, g_ref, e_ref, wes, wen, wee, b_ref, acc_ref):
    i = pl.program_id(0)
    bf16 = jnp.bfloat16
    g2 = g_ref[...].reshape(R2, D).astype(bf16)
    e2 = jnp.transpose(e_ref[...].astype(bf16), (2, 0, 1)).reshape(R2, DE)
    zs = jnp.dot(n_ref[...].astype(bf16), wes[...],
                 preferred_element_type=jnp.float32)
    zn = jnp.dot(g2, wen[...], preferred_element_type=jnp.float32)
    ze = jnp.dot(e2, wee[...], preferred_element_type=jnp.float32)
    e3 = zs[:, None, :] + (zn + ze).reshape(BN2, L, DE) + b_ref[0, :]
    e3 = jnp.where(_node_mask(i, (BN2, 1, 1), 0), e3, 0.0)
    s = jnp.sum(e3, axis=(0, 1))
    ss = jnp.sum(e3 * e3, axis=(0, 1))
    st = jnp.concatenate([s[None, :], ss[None, :]], axis=0)

    @pl.when(i == 0)
    def _():
        acc_ref[...] = jnp.zeros_like(acc_ref)

    acc_ref[...] += st


def _t3_body(n_ref, g_ref, e_ref, acc1, wes, wen, wee, eb, w1s, w1n, w1e, b1,
             wvs, wvn, wve, bv, w2b, b2, exp_ref, eo_ref, w_ref, acc_ref):
    i = pl.program_id(0)
    bf16 = jnp.bfloat16
    g2 = g_ref[...].reshape(R2, D).astype(bf16)
    nblk = n_ref[...].astype(bf16)
    e3d = jnp.transpose(e_ref[...].astype(bf16), (2, 0, 1))
    e2 = e3d.reshape(R2, DE)
    zs = jnp.dot(nblk, wes[...], preferred_element_type=jnp.float32)
    zn = jnp.dot(g2, wen[...], preferred_element_type=jnp.float32)
    ze = jnp.dot(e2, wee[...], preferred_element_type=jnp.float32)
    ep3 = zs[:, None, :] + (zn + ze).reshape(BN2, L, DE) + eb[0, :]
    m1 = acc1[0, :] / NL
    rs1 = lax.rsqrt(acc1[1, :] / NL - m1 * m1 + EPS)
    eo3 = _softplus(e3d.astype(jnp.float32) + (ep3 - m1) * rs1)
    eo_ref[...] = jnp.transpose(eo3.astype(bf16), (1, 2, 0)
                                ).astype(jnp.float32)
    eo2 = eo3.reshape(R2, DE).astype(bf16)
    ps = jnp.dot(nblk, w1s[...], preferred_element_type=jnp.float32)
    vs = jnp.dot(nblk, wvs[...], preferred_element_type=jnp.float32)
    hid3 = _softplus(
        (jnp.dot(g2, w1n[...], preferred_element_type=jnp.float32)
         + jnp.dot(eo2, w1e[...], preferred_element_type=jnp.float32)
         ).reshape(BN2, L, H * A) + ps[:, None, :] + b1[0, :])
    lg = (jnp.dot(hid3.reshape(R2, H * A).astype(bf16), w2b[...],
                  preferred_element_type=jnp.float32) + b2[0, :]
          ).reshape(BN2, L, H)
    mx = jnp.max(lg, axis=1, keepdims=True)
    ex = jnp.exp(lg - mx)
    a3 = ex / jnp.sum(ex, axis=1, keepdims=True)
    aexp = jnp.dot(a3.reshape(R2, H).astype(bf16), exp_ref[...],
                   preferred_element_type=jnp.float32).reshape(BN2, L, H * A)
    v3 = (jnp.dot(g2, wvn[...], preferred_element_type=jnp.float32)
          + jnp.dot(eo2, wve[...], preferred_element_type=jnp.float32)
          ).reshape(BN2, L, H * A) + vs[:, None, :] + bv[0, :]
    w3 = jnp.where(_node_mask(i, (BN2, 1, 1), 0), aexp * v3, 0.0)
    w_ref[...] = w3.astype(jnp.bfloat16)
    s = jnp.sum(w3, axis=(0, 1))
    ss = jnp.sum(w3 * w3, axis=(0, 1))
    st = jnp.concatenate([s[None, :], ss[None, :]], axis=0)

    @pl.when(i == 0)
    def _():
        acc_ref[...] = jnp.zeros_like(acc_ref)

    acc_ref[...] += st


def _t4_body(w_ref, acc3, cat_ref, acc_ref):
    i = pl.program_id(0)
    m = acc3[0, :] / NL
    rs = lax.rsqrt(acc3[1, :] / NL - m * m + EPS)
    hv = _softplus((w_ref[...].astype(jnp.float32) - m) * rs)
    cat = jnp.sum(hv, axis=1)
    cat_ref[...] = cat
    s = jnp.sum(cat, axis=0)
    ss = jnp.sum(cat * cat, axis=0)
    st = jnp.concatenate([s[None, :], ss[None, :]], axis=0)

    @pl.when(i == 0)
    def _():
        acc_ref[...] = jnp.zeros_like(acc_ref)

    acc_ref[...] += st


def _t5_body(n_ref, cat_ref, acc4, out_ref):
    m = acc4[0, :] / N
    rs = lax.rsqrt(acc4[1, :] / N - m * m + EPS)
    out_ref[...] = n_ref[...] + (cat_ref[...] - m) * rs


def _full(shape):
    nd = len(shape)
    return pl.BlockSpec(shape, lambda i: (0,) * nd)


_SEQ = pltpu.CompilerParams(dimension_semantics=("arbitrary",))


def _tc_pipeline(node_features, gathered3, edge_features, edge_W, edge_b,
                 att_W1, att_b1, att_W2, att_b2, val_W, val_b):
    f32 = jnp.float32
    bf16 = jnp.bfloat16
    wes = edge_W[:D].astype(bf16)
    wen = edge_W[D:2 * D].astype(bf16)
    wee = edge_W[2 * D:].astype(bf16)
    w1cat = jnp.transpose(att_W1, (1, 0, 2)).reshape(2 * D + DE, H * A)
    w1s = w1cat[:D].astype(bf16)
    w1n = w1cat[D:2 * D].astype(bf16)
    w1e = w1cat[2 * D:].astype(bf16)
    wvcat = jnp.transpose(val_W, (1, 0, 2)).reshape(2 * D + DE, H * A)
    wvs = wvcat[:D].astype(bf16)
    wvn = wvcat[D:2 * D].astype(bf16)
    wve = wvcat[2 * D:].astype(bf16)
    b1 = att_b1.reshape(1, H * A)
    bv = val_b.reshape(1, H * A)
    w2b = jax.scipy.linalg.block_diag(
        *[att_W2[hh] for hh in range(H)]).astype(bf16)
    b2 = att_b2.reshape(1, H)
    expander = jnp.repeat(jnp.eye(H, dtype=bf16), A, axis=1)
    eb = edge_b.reshape(1, DE)

    nspec = pl.BlockSpec((BN_, D), lambda i: (i, 0))
    gspec = pl.BlockSpec((BN_, L, D), lambda i: (i, 0, 0))
    nspec2 = pl.BlockSpec((BN2, D), lambda i: (i, 0))
    gspec2 = pl.BlockSpec((BN2, L, D), lambda i: (i, 0, 0))
    espec_t = pl.BlockSpec((L, DE, BN2), lambda i: (0, 0, i))
    ef_t = jnp.transpose(edge_features, (1, 2, 0))

    acc1 = pl.pallas_call(
        _t1_body,
        grid=(NBLK2,),
        in_specs=[nspec2, gspec2, espec_t, _full((D, DE)), _full((D, DE)),
                  _full((DE, DE)), _full((1, DE))],
        compiler_params=_SEQ,
        out_specs=_full((2, DE)),
        out_shape=jax.ShapeDtypeStruct((2, DE), f32),
    )(node_features, gathered3, ef_t, wes, wen, wee, eb)

    edge_out_t, weighted, acc3 = pl.pallas_call(
        _t3_body,
        grid=(NBLK2,),
        in_specs=[nspec2, gspec2, espec_t, _full((2, DE)),
                  _full((D, DE)), _full((D, DE)), _full((DE, DE)),
                  _full((1, DE)),
                  _full((D, H * A)), _full((D, H * A)), _full((DE, H * A)),
                  _full((1, H * A)),
                  _full((D, H * A)), _full((D, H * A)), _full((DE, H * A)),
                  _full((1, H * A)),
                  _full((H * A, H)), _full((1, H)), _full((H, H * A))],
        out_specs=[espec_t, gspec2, _full((2, H * A))],
        out_shape=[jax.ShapeDtypeStruct((L, DE, N), f32),
                   jax.ShapeDtypeStruct((N, L, H * A), jnp.bfloat16),
                   jax.ShapeDtypeStruct((2, H * A), f32)],
        compiler_params=_SEQ,
    )(node_features, gathered3, ef_t, acc1, wes, wen, wee, eb,
      w1s, w1n, w1e, b1, wvs, wvn, wve, bv, w2b, b2, expander)
    edge_out = jnp.transpose(edge_out_t, (2, 0, 1))

    cat, acc4 = pl.pallas_call(
        _t4_body,
        grid=(NBLK,),
        in_specs=[gspec, _full((2, H * A))],
        out_specs=[nspec, _full((2, H * A))],
        out_shape=[jax.ShapeDtypeStruct((N, H * A), f32),
                   jax.ShapeDtypeStruct((2, H * A), f32)],
        compiler_params=_SEQ,
    )(weighted, acc3)

    n5spec = pl.BlockSpec((1000, D), lambda i: (i, 0))
    atom_out = pl.pallas_call(
        _t5_body,
        grid=(10,),
        in_specs=[n5spec, n5spec, _full((2, H * A))],
        out_specs=n5spec,
        out_shape=jax.ShapeDtypeStruct((N, D), f32),
        compiler_params=_SEQ,
    )(node_features, cat, acc4)

    return atom_out, edge_out


def kernel(node_features, edge_features, neighbor_indices, neighbor_masks,
           h, c, edge_W, edge_b, att_W1, att_b1, att_W2, att_b2, val_W,
           val_b):
    idx2d = neighbor_indices.astype(jnp.int32).reshape(NW, NCH, GC)
    gathered3 = _sc_gather(node_features, idx2d).reshape(N, L, D)
    atom_out, edge_out = _tc_pipeline(
        node_features, gathered3, edge_features, edge_W, edge_b,
        att_W1, att_b1, att_W2, att_b2, val_W, val_b)
    return (atom_out, edge_out, h, c)
